# baseline (speedup 1.0000x reference)
import functools

import jax
import jax.numpy as jnp
from jax import lax
from jax.experimental import pallas as pl
from jax.experimental.pallas import tpu as pltpu
from jax.experimental.pallas import tpu_sc as plsc

_NC = 2
_NS = 16
_NW = _NC * _NS


def _uv_sparsecore(uid, iid, ue, ie):
    B = uid.shape[0]
    bpw = B // _NW
    nidx = bpw // 128

    mesh = plsc.VectorSubcoreMesh(
        core_axis_name="c", subcore_axis_name="s",
        num_cores=_NC, num_subcores=_NS)

    @functools.partial(
        pl.kernel,
        out_type=(jax.ShapeDtypeStruct((B,), jnp.float32),
                  jax.ShapeDtypeStruct((B,), jnp.float32)),
        mesh=mesh,
        compiler_params=pltpu.CompilerParams(
            use_tc_tiling_on_sc=False, needs_layout_passes=False,
            skip_device_barrier=True),
        scratch_types=[
            pltpu.VMEM((nidx, 128), jnp.int32),
            pltpu.VMEM((nidx, 128), jnp.int32),
            pltpu.VMEM((nidx, 128), jnp.float32),
            pltpu.VMEM((nidx, 128), jnp.float32),
            pltpu.SemaphoreType.DMA,
            pltpu.SemaphoreType.DMA,
        ],
    )
    def uv_kernel(uid_hbm, iid_hbm, ue_hbm, ie_hbm, u_hbm, v_hbm,
                  uid_v, iid_v, u_v, v_v, sem0, sem1):
        wid = lax.axis_index("s") * _NC + lax.axis_index("c")
        base = wid * bpw
        idx_copies = []
        for j in range(nidx):
            idx_copies.append(pltpu.async_copy(
                uid_hbm.at[pl.ds(base + 128 * j, 128)], uid_v.at[j], sem0))
            idx_copies.append(pltpu.async_copy(
                iid_hbm.at[pl.ds(base + 128 * j, 128)], iid_v.at[j], sem0))
        for cp in idx_copies:
            cp.wait()
        emb_copies = []
        for j in range(nidx):
            emb_copies.append(pltpu.async_copy(
                ue_hbm.at[uid_v.at[j]], u_v.at[j], sem1))
            emb_copies.append(pltpu.async_copy(
                ie_hbm.at[iid_v.at[j]], v_v.at[j], sem1))
        for cp in emb_copies:
            cp.wait()
        out_copies = []
        for j in range(nidx):
            out_copies.append(pltpu.async_copy(
                u_v.at[j], u_hbm.at[pl.ds(base + 128 * j, 128)], sem0))
            out_copies.append(pltpu.async_copy(
                v_v.at[j], v_hbm.at[pl.ds(base + 128 * j, 128)], sem0))
        for cp in out_copies:
            cp.wait()

    return uv_kernel(uid, iid, ue, ie)


def _flatten_body(src_ref, dst_ref, sem):
    pltpu.make_async_copy(src_ref.at[0], dst_ref, sem).start()
    pltpu.make_async_copy(src_ref.at[0], dst_ref, sem).wait()


def _flatten_tc(x2d):
    n = x2d.shape[1]
    return pl.pallas_call(
        _flatten_body,
        in_specs=[pl.BlockSpec(memory_space=pl.ANY)],
        out_specs=pl.BlockSpec(memory_space=pl.ANY),
        out_shape=jax.ShapeDtypeStruct((n,), x2d.dtype),
        scratch_shapes=[pltpu.SemaphoreType.DMA],
    )(x2d)


def _matvec_body(fvt_ref, w_ref, o_ref):
    x = fvt_ref[...]
    w = w_ref[...]
    lin = lax.dot_general(w, x, (((1,), (0,)), ((), ())),
                          preferred_element_type=jnp.float32)
    o_ref[...] = lin.reshape(lin.shape[1])


def _matvec_tc(fvt, wt):
    F, B = fvt.shape
    bt = 4096
    return pl.pallas_call(
        _matvec_body,
        grid=(B // bt,),
        in_specs=[
            pl.BlockSpec((F, bt), lambda i: (0, i)),
            pl.BlockSpec((1, F), lambda i: (0, 0)),
        ],
        out_specs=pl.BlockSpec((bt,), lambda i: (i,)),
        out_shape=jax.ShapeDtypeStruct((B,), jnp.float32),
    )(fvt, wt)


def _combine_body(lin_ref, u_ref, v_ref, b_ref, o_ref):
    z = lin_ref[...] + u_ref[...] + v_ref[...] + b_ref[0]
    o_ref[...] = 1.0 / (1.0 + jnp.exp(-z))


def _combine_tc(lin, u, v, b):
    B = lin.shape[0]
    return pl.pallas_call(
        _combine_body,
        in_specs=[
            pl.BlockSpec((B,), lambda: (0,)),
            pl.BlockSpec((B,), lambda: (0,)),
            pl.BlockSpec((B,), lambda: (0,)),
            pl.BlockSpec(memory_space=pltpu.SMEM),
        ],
        out_specs=pl.BlockSpec((B,), lambda: (0,)),
        out_shape=jax.ShapeDtypeStruct((B,), jnp.float32),
    )(lin, u, v, b)


def kernel(feature_vector, user_emb, item_emb, W, b):
    B, F = feature_vector.shape
    uid = feature_vector[:, 0].astype(jnp.int32)
    iid = feature_vector[:, 1].astype(jnp.int32)
    ue_flat = _flatten_tc(user_emb.T)
    ie_flat = _flatten_tc(item_emb.T)
    u, v = _uv_sparsecore(uid, iid, ue_flat, ie_flat)
    fvt = feature_vector.T
    wt = jnp.concatenate([jnp.zeros((1, 2), W.dtype), W.T], axis=1)
    lin = _matvec_tc(fvt, wt)
    out = _combine_tc(lin, u, v, b)
    return out.reshape(B, 1)


# speedup vs baseline: 2.4197x; 2.4197x over previous
import functools

import jax
import jax.numpy as jnp
from jax import lax
from jax.experimental import pallas as pl
from jax.experimental.pallas import tpu as pltpu
from jax.experimental.pallas import tpu_sc as plsc

_NC = 2
_NS = 16
_NW = _NC * _NS


def _uv_sparsecore(uid, iid, ue, ie):
    B = uid.shape[0]
    bpw = B // _NW
    nidx = bpw // 128

    mesh = plsc.VectorSubcoreMesh(
        core_axis_name="c", subcore_axis_name="s",
        num_cores=_NC, num_subcores=_NS)

    @functools.partial(
        pl.kernel,
        out_type=(jax.ShapeDtypeStruct((B,), jnp.float32),
                  jax.ShapeDtypeStruct((B,), jnp.float32)),
        mesh=mesh,
        compiler_params=pltpu.CompilerParams(
            use_tc_tiling_on_sc=False, needs_layout_passes=False,
            skip_device_barrier=True),
        scratch_types=[
            pltpu.VMEM((nidx, 128), jnp.int32),
            pltpu.VMEM((nidx, 128), jnp.int32),
            pltpu.VMEM((nidx, 128), jnp.float32),
            pltpu.VMEM((nidx, 128), jnp.float32),
            pltpu.SemaphoreType.DMA,
            pltpu.SemaphoreType.DMA,
        ],
    )
    def uv_kernel(uid_hbm, iid_hbm, ue_hbm, ie_hbm, u_hbm, v_hbm,
                  uid_v, iid_v, u_v, v_v, sem0, sem1):
        wid = lax.axis_index("s") * _NC + lax.axis_index("c")
        base = wid * bpw
        idx_copies = []
        for j in range(nidx):
            idx_copies.append(pltpu.async_copy(
                uid_hbm.at[pl.ds(base + 128 * j, 128)], uid_v.at[j], sem0))
            idx_copies.append(pltpu.async_copy(
                iid_hbm.at[pl.ds(base + 128 * j, 128)], iid_v.at[j], sem0))
        for cp in idx_copies:
            cp.wait()
        emb_copies = []
        for j in range(nidx):
            emb_copies.append(pltpu.async_copy(
                ue_hbm.at[0].at[uid_v.at[j]], u_v.at[j], sem1))
            emb_copies.append(pltpu.async_copy(
                ie_hbm.at[0].at[iid_v.at[j]], v_v.at[j], sem1))
        for cp in emb_copies:
            cp.wait()
        out_copies = []
        for j in range(nidx):
            out_copies.append(pltpu.async_copy(
                u_v.at[j], u_hbm.at[pl.ds(base + 128 * j, 128)], sem0))
            out_copies.append(pltpu.async_copy(
                v_v.at[j], v_hbm.at[pl.ds(base + 128 * j, 128)], sem0))
        for cp in out_copies:
            cp.wait()

    return uv_kernel(uid, iid, ue, ie)


def _matvec_body(fvt_ref, w_ref, o_ref):
    x = fvt_ref[...]
    w = w_ref[...]
    lin = lax.dot_general(w, x, (((1,), (0,)), ((), ())),
                          preferred_element_type=jnp.float32)
    o_ref[...] = lin.reshape(lin.shape[1])


def _matvec_tc(fvt, wt):
    F, B = fvt.shape
    bt = 4096
    return pl.pallas_call(
        _matvec_body,
        grid=(B // bt,),
        in_specs=[
            pl.BlockSpec((F, bt), lambda i: (0, i)),
            pl.BlockSpec((1, F), lambda i: (0, 0)),
        ],
        out_specs=pl.BlockSpec((bt,), lambda i: (i,)),
        out_shape=jax.ShapeDtypeStruct((B,), jnp.float32),
    )(fvt, wt)


def _combine_body(lin_ref, u_ref, v_ref, b_ref, o_ref):
    z = lin_ref[...] + u_ref[...] + v_ref[...] + b_ref[0]
    o_ref[...] = 1.0 / (1.0 + jnp.exp(-z))


def _combine_tc(lin, u, v, b):
    B = lin.shape[0]
    return pl.pallas_call(
        _combine_body,
        in_specs=[
            pl.BlockSpec((B,), lambda: (0,)),
            pl.BlockSpec((B,), lambda: (0,)),
            pl.BlockSpec((B,), lambda: (0,)),
            pl.BlockSpec(memory_space=pltpu.SMEM),
        ],
        out_specs=pl.BlockSpec((B,), lambda: (0,)),
        out_shape=jax.ShapeDtypeStruct((B,), jnp.float32),
    )(lin, u, v, b)


def kernel(feature_vector, user_emb, item_emb, W, b):
    B, F = feature_vector.shape
    uid = feature_vector[:, 0].astype(jnp.int32)
    iid = feature_vector[:, 1].astype(jnp.int32)
    u, v = _uv_sparsecore(uid, iid, user_emb.T, item_emb.T)
    fvt = feature_vector.T
    wt = jnp.concatenate([jnp.zeros((1, 2), W.dtype), W.T], axis=1)
    lin = _matvec_tc(fvt, wt)
    out = _combine_tc(lin, u, v, b)
    return out.reshape(B, 1)


# speedup vs baseline: 2.4320x; 1.0051x over previous
import functools

import jax
import jax.numpy as jnp
from jax import lax
from jax.experimental import pallas as pl
from jax.experimental.pallas import tpu as pltpu
from jax.experimental.pallas import tpu_sc as plsc

_NC = 2
_NS = 16
_NW = _NC * _NS


def _uv_sparsecore(uid, iid, ue, ie):
    B = uid.shape[0]
    bpw = B // _NW
    nidx = bpw // 128

    mesh = plsc.VectorSubcoreMesh(
        core_axis_name="c", subcore_axis_name="s",
        num_cores=_NC, num_subcores=_NS)

    @functools.partial(
        pl.kernel,
        out_type=(jax.ShapeDtypeStruct((B,), jnp.float32),
                  jax.ShapeDtypeStruct((B,), jnp.float32)),
        mesh=mesh,
        compiler_params=pltpu.CompilerParams(
            use_tc_tiling_on_sc=False, needs_layout_passes=False,
            skip_device_barrier=True),
        scratch_types=[
            pltpu.VMEM((nidx, 128), jnp.int32),
            pltpu.VMEM((nidx, 128), jnp.int32),
            pltpu.VMEM((nidx, 128), jnp.float32),
            pltpu.VMEM((nidx, 128), jnp.float32),
            pltpu.SemaphoreType.DMA,
            pltpu.SemaphoreType.DMA,
        ],
    )
    def uv_kernel(uid_hbm, iid_hbm, ue_hbm, ie_hbm, u_hbm, v_hbm,
                  uid_v, iid_v, u_v, v_v, sem0, sem1):
        wid = lax.axis_index("s") * _NC + lax.axis_index("c")
        base = wid * bpw
        idx_copies = []
        for j in range(nidx):
            idx_copies.append(pltpu.async_copy(
                uid_hbm.at[pl.ds(base + 128 * j, 128)], uid_v.at[j], sem0))
            idx_copies.append(pltpu.async_copy(
                iid_hbm.at[pl.ds(base + 128 * j, 128)], iid_v.at[j], sem0))
        for cp in idx_copies:
            cp.wait()
        emb_copies = []
        for j in range(nidx):
            emb_copies.append(pltpu.async_copy(
                ue_hbm.at[0].at[uid_v.at[j]], u_v.at[j], sem1))
            emb_copies.append(pltpu.async_copy(
                ie_hbm.at[0].at[iid_v.at[j]], v_v.at[j], sem1))
        for cp in emb_copies:
            cp.wait()
        out_copies = []
        for j in range(nidx):
            out_copies.append(pltpu.async_copy(
                u_v.at[j], u_hbm.at[pl.ds(base + 128 * j, 128)], sem0))
            out_copies.append(pltpu.async_copy(
                v_v.at[j], v_hbm.at[pl.ds(base + 128 * j, 128)], sem0))
        for cp in out_copies:
            cp.wait()

    return uv_kernel(uid, iid, ue, ie)


def _matvec_body(fvt_ref, w_ref, o_ref):
    x = fvt_ref[...]
    w = w_ref[...]
    lin = lax.dot_general(w, x, (((1,), (0,)), ((), ())),
                          preferred_element_type=jnp.float32)
    o_ref[...] = lin.reshape(lin.shape[1])


def _matvec_tc(fvt, wt):
    F, B = fvt.shape
    bt = 8192
    return pl.pallas_call(
        _matvec_body,
        grid=(B // bt,),
        in_specs=[
            pl.BlockSpec((F, bt), lambda i: (0, i)),
            pl.BlockSpec((1, F), lambda i: (0, 0)),
        ],
        out_specs=pl.BlockSpec((bt,), lambda i: (i,)),
        out_shape=jax.ShapeDtypeStruct((B,), jnp.float32),
    )(fvt, wt)


def _combine_body(lin_ref, u_ref, v_ref, b_ref, o_ref):
    z = lin_ref[...] + u_ref[...] + v_ref[...] + b_ref[0]
    o_ref[...] = 1.0 / (1.0 + jnp.exp(-z))


def _combine_tc(lin, u, v, b):
    B = lin.shape[0]
    return pl.pallas_call(
        _combine_body,
        in_specs=[
            pl.BlockSpec((B,), lambda: (0,)),
            pl.BlockSpec((B,), lambda: (0,)),
            pl.BlockSpec((B,), lambda: (0,)),
            pl.BlockSpec(memory_space=pltpu.SMEM),
        ],
        out_specs=pl.BlockSpec((B,), lambda: (0,)),
        out_shape=jax.ShapeDtypeStruct((B,), jnp.float32),
    )(lin, u, v, b)


def kernel(feature_vector, user_emb, item_emb, W, b):
    B, F = feature_vector.shape
    uid = feature_vector[:, 0].astype(jnp.int32)
    iid = feature_vector[:, 1].astype(jnp.int32)
    u, v = _uv_sparsecore(uid, iid, user_emb.T, item_emb.T)
    fvt = feature_vector.T
    wt = jnp.concatenate([jnp.zeros((1, 2), W.dtype), W.T], axis=1)
    lin = _matvec_tc(fvt, wt)
    out = _combine_tc(lin, u, v, b)
    return out.reshape(B, 1)
